# baseline (speedup 1.0000x reference)
import functools

import jax
import jax.numpy as jnp
from jax import lax
from jax.experimental import pallas as pl
from jax.experimental.pallas import tpu as pltpu
from jax.experimental.pallas import tpu_sc as plsc

N = 10000
E = 320000
DIN = 128
DH = 256
HALF = DH // 2

NTILES = 16
EB = 128
CH = 16
NB = 160
NC = NB // CH
SC_DT = jnp.float32
EPT = NB * EB
EPAD = NTILES * EPT
ACC_ROWS = N + 8
NSLOT = 2
RPT = 624
RPT_LAST = N - 15 * RPT


def _sc_segment_sum(xi, srcb, dstb):
    mesh = plsc.VectorSubcoreMesh(core_axis_name="c", subcore_axis_name="s")

    @functools.partial(
        pl.kernel,
        out_type=jax.ShapeDtypeStruct((2, N, HALF), SC_DT),
        mesh=mesh,
        scratch_types=[
            pltpu.VMEM((CH, EB), jnp.int32),
            pltpu.VMEM((CH, EB), jnp.int32),
            pltpu.VMEM((CH, EB), jnp.int32),
            pltpu.VMEM((CH, EB), jnp.int32),
            [pltpu.VMEM((EB, HALF), SC_DT)] * NSLOT,
            [pltpu.SemaphoreType.DMA] * NSLOT,
            pltpu.SemaphoreType.DMA,
            pltpu.VMEM_SHARED((ACC_ROWS, HALF), SC_DT),
        ],
    )
    def k(xi_hbm, srcb_hbm, dstb_hbm, out_hbm,
          src0, src1, dst0, dst1, rows, semg, semi, acc):
        c = lax.axis_index("c")
        s = lax.axis_index("s")
        xi_c = xi_hbm.at[c]
        srcb_t = srcb_hbm.at[s]
        dstb_t = dstb_hbm.at[s]

        @pl.when(s < NTILES - 1)
        def _():
            pltpu.sync_copy(xi_c.at[pl.ds(s * RPT, RPT)],
                            acc.at[pl.ds(s * RPT, RPT)])

        @pl.when(s == NTILES - 1)
        def _():
            pltpu.sync_copy(xi_c.at[pl.ds(15 * RPT, RPT_LAST)],
                            acc.at[pl.ds(15 * RPT, RPT_LAST)])

        plsc.subcore_barrier()

        pltpu.sync_copy(srcb_t.at[pl.ds(0, CH)], src0)
        pltpu.sync_copy(dstb_t.at[pl.ds(0, CH)], dst0)
        pltpu.async_copy(xi_c.at[src0.at[0]], rows[0], semg[0])
        pltpu.async_copy(xi_c.at[src0.at[1]], rows[1], semg[1])

        def wait_gather(k):
            pltpu.make_async_copy(xi_c.at[pl.ds(0, EB)], rows[k], semg[k]).wait()

        def do_chunk(qnext, sbuf, dbuf, sbuf_next, dbuf_next):
            off = pl.multiple_of(qnext * CH, CH)
            for b in range(CH):
                k = b % NSLOT
                if b == 0:
                    pltpu.async_copy(srcb_t.at[pl.ds(off, CH)], sbuf_next, semi)
                    pltpu.async_copy(dstb_t.at[pl.ds(off, CH)], dbuf_next, semi)
                if b == CH - 2:
                    pltpu.make_async_copy(srcb_t.at[pl.ds(0, CH)], sbuf_next, semi).wait()
                    pltpu.make_async_copy(dstb_t.at[pl.ds(0, CH)], dbuf_next, semi).wait()
                wait_gather(k)
                pltpu.sync_copy(rows[k], acc.at[dbuf.at[b]], add=True)
                if b + 2 < CH:
                    pltpu.async_copy(xi_c.at[sbuf.at[b + 2]], rows[k], semg[k])
                else:
                    pltpu.async_copy(xi_c.at[sbuf_next.at[b + 2 - CH]], rows[k], semg[k])

        def body(g, carry):
            q0 = 2 * g
            do_chunk(q0 + 1, src0, dst0, src1, dst1)
            do_chunk(lax.rem(q0 + 2, NC), src1, dst1, src0, dst0)
            return carry

        lax.fori_loop(0, NC // 2, body, 0)

        wait_gather(0)
        wait_gather(1)

        plsc.subcore_barrier()

        @pl.when(s < NTILES - 1)
        def _():
            pltpu.sync_copy(acc.at[pl.ds(s * RPT, RPT)],
                            out_hbm.at[c].at[pl.ds(s * RPT, RPT)])

        @pl.when(s == NTILES - 1)
        def _():
            pltpu.sync_copy(acc.at[pl.ds(15 * RPT, RPT_LAST)],
                            out_hbm.at[c].at[pl.ds(15 * RPT, RPT_LAST)])

    return k(xi, srcb, dstb)


_RB = 10000
_GRID = N // _RB


def _head_body(x_ref, scale_ref, beta_ref, w1t_ref, b1_ref, wc1t_ref, out_ref):
    h = x_ref[...] * scale_ref[...] + beta_ref[...]
    t = jnp.dot(h, w1t_ref[...], preferred_element_type=jnp.float32) + b1_ref[...]
    t = jnp.maximum(t, 0.0)
    xi = jnp.dot(t, wc1t_ref[...], preferred_element_type=jnp.float32).astype(SC_DT)
    out_ref[0] = xi[:, :HALF]
    out_ref[1] = xi[:, HALF:]


def _mlp_in(x, scale, beta, w1t, b1, wc1t):
    return pl.pallas_call(
        _head_body,
        grid=(_GRID,),
        in_specs=[
            pl.BlockSpec((_RB, DIN), lambda i: (i, 0)),
            pl.BlockSpec((1, DIN), lambda i: (0, 0)),
            pl.BlockSpec((1, DIN), lambda i: (0, 0)),
            pl.BlockSpec((DIN, DH), lambda i: (0, 0)),
            pl.BlockSpec((1, DH), lambda i: (0, 0)),
            pl.BlockSpec((DH, DH), lambda i: (0, 0)),
        ],
        out_specs=pl.BlockSpec((2, _RB, HALF), lambda i: (0, i, 0)),
        out_shape=jax.ShapeDtypeStruct((2, N, HALF), SC_DT),
    )(x, scale, beta, w1t, b1, wc1t)


def _mid_body(agg_ref, bias_ref, wct_ref, out_ref):
    h = jnp.concatenate([agg_ref[0], agg_ref[1]], axis=1).astype(jnp.float32)
    h = jnp.maximum(h + bias_ref[...], 0.0)
    xi = jnp.dot(h, wct_ref[...], preferred_element_type=jnp.float32).astype(SC_DT)
    out_ref[0] = xi[:, :HALF]
    out_ref[1] = xi[:, HALF:]


def _mlp_mid(agg, bias, wct):
    return pl.pallas_call(
        _mid_body,
        grid=(_GRID,),
        in_specs=[
            pl.BlockSpec((2, _RB, HALF), lambda i: (0, i, 0)),
            pl.BlockSpec((1, DH), lambda i: (0, 0)),
            pl.BlockSpec((DH, DH), lambda i: (0, 0)),
        ],
        out_specs=pl.BlockSpec((2, _RB, HALF), lambda i: (0, i, 0)),
        out_shape=jax.ShapeDtypeStruct((2, N, HALF), SC_DT),
    )(agg, bias, wct)


def _tail_body(agg_ref, bias_ref, hw_ref, hb_ref, out_ref, acc_ref):
    i = pl.program_id(0)
    h = jnp.concatenate([agg_ref[0], agg_ref[1]], axis=1).astype(jnp.float32)
    h = jnp.maximum(h + bias_ref[...], 0.0)
    part = jnp.sum(h, axis=0, keepdims=True)

    @pl.when(i == 0)
    def _():
        acc_ref[...] = part

    @pl.when(i > 0)
    def _():
        acc_ref[...] += part

    @pl.when(i == pl.num_programs(0) - 1)
    def _():
        g = acc_ref[...] * (1.0 / N)
        out_ref[...] = jnp.sum(g * hw_ref[...], axis=1, keepdims=True) + hb_ref[...]


def _mlp_tail(agg, bias, head_w, head_b):
    return pl.pallas_call(
        _tail_body,
        grid=(_GRID,),
        in_specs=[
            pl.BlockSpec((2, _RB, HALF), lambda i: (0, i, 0)),
            pl.BlockSpec((1, DH), lambda i: (0, 0)),
            pl.BlockSpec((1, DH), lambda i: (0, 0)),
            pl.BlockSpec((1, 1), lambda i: (0, 0)),
        ],
        out_specs=pl.BlockSpec((1, 1), lambda i: (0, 0)),
        out_shape=jax.ShapeDtypeStruct((1, 1), jnp.float32),
        scratch_shapes=[pltpu.VMEM((1, DH), jnp.float32)],
    )(agg, bias, head_w, head_b)


def kernel(x, edge_index, bn_gamma, bn_beta, lin1_W, lin1_b, conv1_Win,
           conv1_bias, conv2_Win, conv2_bias, head_W, head_b):
    scale = (bn_gamma * (1.0 / jnp.sqrt(1.0 + 1e-5))).reshape(1, DIN)
    beta = bn_beta.reshape(1, DIN)
    w1t = lin1_W.T
    b1 = lin1_b.reshape(1, DH)
    wc1t = conv1_Win.T
    wc2t = conv2_Win.T
    bias1 = conv1_bias.reshape(1, DH)
    bias2 = conv2_bias.reshape(1, DH)
    hb = head_b.reshape(1, 1)

    src = edge_index[0]
    dst = edge_index[1]
    pad = EPAD - E
    srcp = jnp.concatenate([src, jnp.zeros((pad,), jnp.int32)])
    dstp = jnp.concatenate([dst, jnp.full((pad,), N, jnp.int32)])
    srcb = srcp.reshape(NTILES, NB, EB)
    dstb = dstp.reshape(NTILES, NB, EB)

    xi1 = _mlp_in(x, scale, beta, w1t, b1, wc1t)
    agg1 = _sc_segment_sum(xi1, srcb, dstb)
    xi2 = _mlp_mid(agg1, bias1, wc2t)
    agg2 = _sc_segment_sum(xi2, srcb, dstb)
    return _mlp_tail(agg2, bias2, head_W, hb)


# speedup vs baseline: 1.0043x; 1.0043x over previous
import functools

import jax
import jax.numpy as jnp
from jax import lax
from jax.experimental import pallas as pl
from jax.experimental.pallas import tpu as pltpu
from jax.experimental.pallas import tpu_sc as plsc

N = 10000
E = 320000
DIN = 128
DH = 256
HALF = DH // 2

NTILES = 16
EB = 128
CH = 16
NB = 160
NC = NB // CH
SC_DT = jnp.float32
EPT = NB * EB
EPAD = NTILES * EPT
ACC_ROWS = N + 8
NSLOT = 2
RPT = 624
RPT_LAST = N - 15 * RPT


def _sc_segment_sum(xi, srcb, dstb):
    mesh = plsc.VectorSubcoreMesh(core_axis_name="c", subcore_axis_name="s")

    @functools.partial(
        pl.kernel,
        out_type=jax.ShapeDtypeStruct((2, N, HALF), SC_DT),
        mesh=mesh,
        scratch_types=[
            pltpu.VMEM((CH, EB), jnp.int32),
            pltpu.VMEM((CH, EB), jnp.int32),
            pltpu.VMEM((CH, EB), jnp.int32),
            pltpu.VMEM((CH, EB), jnp.int32),
            [pltpu.VMEM((EB, HALF), SC_DT)] * NSLOT,
            [pltpu.SemaphoreType.DMA] * NSLOT,
            pltpu.SemaphoreType.DMA,
            pltpu.VMEM_SHARED((ACC_ROWS, HALF), SC_DT),
        ],
    )
    def k(xi_hbm, srcb_hbm, dstb_hbm, out_hbm,
          src0, src1, dst0, dst1, rows, semg, semi, acc):
        c = lax.axis_index("c")
        s = lax.axis_index("s")
        xi_c = xi_hbm.at[c]
        srcb_t = srcb_hbm.at[s]
        dstb_t = dstb_hbm.at[s]

        @pl.when(s < NTILES - 1)
        def _():
            pltpu.sync_copy(xi_c.at[pl.ds(s * RPT, RPT)],
                            acc.at[pl.ds(s * RPT, RPT)])

        @pl.when(s == NTILES - 1)
        def _():
            pltpu.sync_copy(xi_c.at[pl.ds(15 * RPT, RPT_LAST)],
                            acc.at[pl.ds(15 * RPT, RPT_LAST)])

        plsc.subcore_barrier()

        pltpu.sync_copy(srcb_t.at[pl.ds(0, CH)], src0)
        pltpu.sync_copy(dstb_t.at[pl.ds(0, CH)], dst0)
        pltpu.async_copy(xi_c.at[src0.at[0]], rows[0], semg[0])
        pltpu.async_copy(xi_c.at[src0.at[1]], rows[1], semg[1])

        def wait_gather(k):
            pltpu.make_async_copy(xi_c.at[pl.ds(0, EB)], rows[k], semg[k]).wait()

        def do_chunk(qnext, sbuf, dbuf, sbuf_next, dbuf_next):
            off = pl.multiple_of(qnext * CH, CH)
            for b in range(CH):
                k = b % NSLOT
                if b == 0:
                    pltpu.async_copy(srcb_t.at[pl.ds(off, CH)], sbuf_next, semi)
                    pltpu.async_copy(dstb_t.at[pl.ds(off, CH)], dbuf_next, semi)
                if b == CH - 2:
                    pltpu.make_async_copy(srcb_t.at[pl.ds(0, CH)], sbuf_next, semi).wait()
                    pltpu.make_async_copy(dstb_t.at[pl.ds(0, CH)], dbuf_next, semi).wait()
                wait_gather(k)
                pltpu.sync_copy(rows[k], acc.at[dbuf.at[b]], add=True)
                if b + 2 < CH:
                    pltpu.async_copy(xi_c.at[sbuf.at[b + 2]], rows[k], semg[k])
                else:
                    pltpu.async_copy(xi_c.at[sbuf_next.at[b + 2 - CH]], rows[k], semg[k])

        def body(g, carry):
            q0 = 2 * g
            do_chunk(q0 + 1, src0, dst0, src1, dst1)
            do_chunk(lax.rem(q0 + 2, NC), src1, dst1, src0, dst0)
            return carry

        lax.fori_loop(0, NC // 2, body, 0)

        wait_gather(0)
        wait_gather(1)

        plsc.subcore_barrier()

        @pl.when(s < NTILES - 1)
        def _():
            pltpu.sync_copy(acc.at[pl.ds(s * RPT, RPT)],
                            out_hbm.at[c].at[pl.ds(s * RPT, RPT)])

        @pl.when(s == NTILES - 1)
        def _():
            pltpu.sync_copy(acc.at[pl.ds(15 * RPT, RPT_LAST)],
                            out_hbm.at[c].at[pl.ds(15 * RPT, RPT_LAST)])

    return k(xi, srcb, dstb)


_RB = 5000
_GRID = N // _RB


def _head_body(x_ref, scale_ref, beta_ref, w1t_ref, b1_ref, wc1t_ref, out_ref):
    h = x_ref[...] * scale_ref[...] + beta_ref[...]
    t = jnp.dot(h, w1t_ref[...], preferred_element_type=jnp.float32) + b1_ref[...]
    t = jnp.maximum(t, 0.0)
    xi = jnp.dot(t, wc1t_ref[...], preferred_element_type=jnp.float32).astype(SC_DT)
    out_ref[0] = xi[:, :HALF]
    out_ref[1] = xi[:, HALF:]


def _mlp_in(x, scale, beta, w1t, b1, wc1t):
    return pl.pallas_call(
        _head_body,
        grid=(_GRID,),
        in_specs=[
            pl.BlockSpec((_RB, DIN), lambda i: (i, 0)),
            pl.BlockSpec((1, DIN), lambda i: (0, 0)),
            pl.BlockSpec((1, DIN), lambda i: (0, 0)),
            pl.BlockSpec((DIN, DH), lambda i: (0, 0)),
            pl.BlockSpec((1, DH), lambda i: (0, 0)),
            pl.BlockSpec((DH, DH), lambda i: (0, 0)),
        ],
        out_specs=pl.BlockSpec((2, _RB, HALF), lambda i: (0, i, 0)),
        out_shape=jax.ShapeDtypeStruct((2, N, HALF), SC_DT),
    )(x, scale, beta, w1t, b1, wc1t)


def _mid_body(agg_ref, bias_ref, wct_ref, out_ref):
    h = jnp.concatenate([agg_ref[0], agg_ref[1]], axis=1).astype(jnp.float32)
    h = jnp.maximum(h + bias_ref[...], 0.0)
    xi = jnp.dot(h, wct_ref[...], preferred_element_type=jnp.float32).astype(SC_DT)
    out_ref[0] = xi[:, :HALF]
    out_ref[1] = xi[:, HALF:]


def _mlp_mid(agg, bias, wct):
    return pl.pallas_call(
        _mid_body,
        grid=(_GRID,),
        in_specs=[
            pl.BlockSpec((2, _RB, HALF), lambda i: (0, i, 0)),
            pl.BlockSpec((1, DH), lambda i: (0, 0)),
            pl.BlockSpec((DH, DH), lambda i: (0, 0)),
        ],
        out_specs=pl.BlockSpec((2, _RB, HALF), lambda i: (0, i, 0)),
        out_shape=jax.ShapeDtypeStruct((2, N, HALF), SC_DT),
    )(agg, bias, wct)


def _tail_body(agg_ref, bias_ref, hw_ref, hb_ref, out_ref, acc_ref):
    i = pl.program_id(0)
    h = jnp.concatenate([agg_ref[0], agg_ref[1]], axis=1).astype(jnp.float32)
    h = jnp.maximum(h + bias_ref[...], 0.0)
    part = jnp.sum(h, axis=0, keepdims=True)

    @pl.when(i == 0)
    def _():
        acc_ref[...] = part

    @pl.when(i > 0)
    def _():
        acc_ref[...] += part

    @pl.when(i == pl.num_programs(0) - 1)
    def _():
        g = acc_ref[...] * (1.0 / N)
        out_ref[...] = jnp.sum(g * hw_ref[...], axis=1, keepdims=True) + hb_ref[...]


def _mlp_tail(agg, bias, head_w, head_b):
    return pl.pallas_call(
        _tail_body,
        grid=(_GRID,),
        in_specs=[
            pl.BlockSpec((2, _RB, HALF), lambda i: (0, i, 0)),
            pl.BlockSpec((1, DH), lambda i: (0, 0)),
            pl.BlockSpec((1, DH), lambda i: (0, 0)),
            pl.BlockSpec((1, 1), lambda i: (0, 0)),
        ],
        out_specs=pl.BlockSpec((1, 1), lambda i: (0, 0)),
        out_shape=jax.ShapeDtypeStruct((1, 1), jnp.float32),
        scratch_shapes=[pltpu.VMEM((1, DH), jnp.float32)],
    )(agg, bias, head_w, head_b)


def kernel(x, edge_index, bn_gamma, bn_beta, lin1_W, lin1_b, conv1_Win,
           conv1_bias, conv2_Win, conv2_bias, head_W, head_b):
    scale = (bn_gamma * (1.0 / jnp.sqrt(1.0 + 1e-5))).reshape(1, DIN)
    beta = bn_beta.reshape(1, DIN)
    w1t = lin1_W.T
    b1 = lin1_b.reshape(1, DH)
    wc1t = conv1_Win.T
    wc2t = conv2_Win.T
    bias1 = conv1_bias.reshape(1, DH)
    bias2 = conv2_bias.reshape(1, DH)
    hb = head_b.reshape(1, 1)

    src = edge_index[0]
    dst = edge_index[1]
    pad = EPAD - E
    srcp = jnp.concatenate([src, jnp.zeros((pad,), jnp.int32)])
    dstp = jnp.concatenate([dst, jnp.full((pad,), N, jnp.int32)])
    srcb = srcp.reshape(NTILES, NB, EB)
    dstb = dstp.reshape(NTILES, NB, EB)

    xi1 = _mlp_in(x, scale, beta, w1t, b1, wc1t)
    agg1 = _sc_segment_sum(xi1, srcb, dstb)
    xi2 = _mlp_mid(agg1, bias1, wc2t)
    agg2 = _sc_segment_sum(xi2, srcb, dstb)
    return _mlp_tail(agg2, bias2, head_W, hb)
